# baseline (speedup 1.0000x reference)
import functools

import jax
import jax.numpy as jnp
from jax import lax
from jax.experimental import pallas as pl
from jax.experimental.pallas import tpu as pltpu
from jax.experimental.pallas import tpu_sc as plsc

_N = 100000
_E = 1600000
_HID = 32
_HH = 16
_NG = 256
_NC = 10
_DIN = 48
_TW = 64

_NB = 5000
_GB = _N // _NB

_NP = 100096
_CH = 128
_ROWS = 12544
_KCH = 16
_RPT = _ROWS // 16
_NPT = _NP // 16
_G = 4
_NGRP = _KCH // _G


def _tables_body(se_ref, ce_ref, pe_ref, ws_ref, wc_ref, wp_ref, t_ref):
  hp = lax.Precision.HIGHEST
  t_ref[0:32, :] = jnp.dot(se_ref[...], ws_ref[...],
                           preferred_element_type=jnp.float32, precision=hp)
  t_ref[32:64, :] = jnp.dot(ce_ref[...], wc_ref[...],
                            preferred_element_type=jnp.float32, precision=hp)
  t_ref[64:192, :] = jnp.dot(pe_ref[...], wp_ref[...],
                             preferred_element_type=jnp.float32, precision=hp)


def _tables(se, ce, pe, ws, wc, wp):
  return pl.pallas_call(
      _tables_body,
      out_shape=jax.ShapeDtypeStruct((192, _TW), jnp.float32),
  )(se, ce, pe, ws, wc, wp)


def _embed_body(x0_ref, x1_ref, x2_ref, t_ref, b1_ref,
                ql_ref, qr_ref, r1_ref):
  x0 = x0_ref[0, 0, :]
  x1 = x1_ref[0, 0, :]
  x2 = x2_ref[0, 0, :]
  i32 = lambda n: lax.broadcasted_iota(jnp.int32, (_NB, n), 1)
  oh = jnp.concatenate([
      (x0[:, None] == i32(32)).astype(jnp.float32),
      (x1[:, None] == i32(32)).astype(jnp.float32),
      (x2[:, None] == i32(128)).astype(jnp.float32),
  ], axis=1)
  qr = jnp.dot(oh, t_ref[...], preferred_element_type=jnp.float32,
               precision=lax.Precision.HIGHEST)
  qr = qr + b1_ref[...]
  ql_ref[...] = qr[:, 0:_HH]
  qr_ref[...] = qr[:, _HH:_HID]
  r1_ref[...] = qr[:, _HID:_TW]


def _embed(x0r, x1r, x2r, t, b1r):
  blk_idx = pl.BlockSpec((1, 1, _NB), lambda i: (i, 0, 0))
  blk_h = pl.BlockSpec((_NB, _HH), lambda i: (i, 0))
  return pl.pallas_call(
      _embed_body,
      grid=(_GB,),
      in_specs=[
          blk_idx, blk_idx, blk_idx,
          pl.BlockSpec((192, _TW), lambda i: (0, 0)),
          pl.BlockSpec((1, _TW), lambda i: (0, 0)),
      ],
      out_specs=(
          blk_h, blk_h,
          pl.BlockSpec((_NB, _HID), lambda i: (i, 0)),
      ),
      out_shape=(
          jax.ShapeDtypeStruct((_NP, _HH), jnp.float32),
          jax.ShapeDtypeStruct((_NP, _HH), jnp.float32),
          jax.ShapeDtypeStruct((_N, _HID), jnp.float32),
      ),
  )(x0r, x1r, x2r, t, b1r)


def _make_edge_pass():
  mesh = plsc.VectorSubcoreMesh(core_axis_name="c", subcore_axis_name="s")

  out_type = [
      jax.ShapeDtypeStruct((_NP, _HH), jnp.float32),
      jax.ShapeDtypeStruct((_NP, _HH), jnp.float32),
  ]
  scratch = [
      pltpu.VMEM((_KCH, _CH), jnp.int32),
      pltpu.VMEM((_KCH, _CH), jnp.int32),
      pltpu.VMEM((3 * _G, _CH, _HH), jnp.float32),
      pltpu.VMEM_SHARED((_NP, _HH), jnp.float32),
      pltpu.SemaphoreType.DMA,
      pltpu.SemaphoreType.DMA,
      pltpu.SemaphoreType.DMA,
      pltpu.SemaphoreType.DMA,
  ]

  def body(ql, qr, srcr, dstr, z16, out_l, out_r,
           sidx, didx, rows, acc, sem_g0, sem_g1, sem_g2, sem_s):
    c = lax.axis_index("c")
    s = lax.axis_index("s")
    sem_g = [sem_g0, sem_g1, sem_g2]

    pltpu.sync_copy(z16.at[pl.ds(s * _NPT, _NPT)],
                    acc.at[pl.ds(s * _NPT, _NPT)])
    plsc.subcore_barrier()

    def run(qtab):
      row0 = s * _RPT

      def fire_gathers(grp, buf):
        for k in range(_G):
          pltpu.async_copy(qtab.at[sidx.at[grp * _G + k]],
                           rows.at[buf * _G + k], sem_g[buf])

      def drain_gathers(buf):
        for k in range(_G):
          pltpu.make_async_copy(qtab.at[sidx.at[k]],
                                rows.at[buf * _G + k], sem_g[buf]).wait()

      def fire_scatters(grp, buf):
        for k in range(_G):
          pltpu.async_copy(rows.at[buf * _G + k],
                           acc.at[didx.at[grp * _G + k]], sem_s, add=True)

      def drain_scatters(n):
        for k in range(n * _G):
          pltpu.make_async_copy(rows.at[k % (3 * _G)],
                                acc.at[didx.at[0]], sem_s).wait()

      def batch_body(b, carry):
        r0 = row0 + b * _KCH
        pltpu.sync_copy(srcr.at[pl.ds(r0, _KCH)], sidx)
        pltpu.sync_copy(dstr.at[pl.ds(r0, _KCH)], didx)
        fire_gathers(0, 0)
        fire_gathers(1, 1)
        fire_gathers(2, 2)
        drain_gathers(0)
        fire_scatters(0, 0)
        drain_gathers(1)
        fire_scatters(1, 1)
        drain_scatters(1)
        fire_gathers(3, 0)
        drain_gathers(2)
        fire_scatters(2, 2)
        drain_gathers(0)
        fire_scatters(3, 0)
        drain_scatters(3)
        return carry

      lax.fori_loop(0, _RPT // _KCH, batch_body, 0)

    @pl.when(c == 0)
    def _():
      run(ql)

    @pl.when(c == 1)
    def _():
      run(qr)

    plsc.subcore_barrier()

    @pl.when(c == 0)
    def _():
      pltpu.sync_copy(acc.at[pl.ds(s * _NPT, _NPT)],
                      out_l.at[pl.ds(s * _NPT, _NPT)])

    @pl.when(c == 1)
    def _():
      pltpu.sync_copy(acc.at[pl.ds(s * _NPT, _NPT)],
                      out_r.at[pl.ds(s * _NPT, _NPT)])

  return pl.kernel(body, mesh=mesh, out_type=out_type,
                   scratch_types=scratch,
                   compiler_params=pltpu.CompilerParams(
                       use_tc_tiling_on_sc=False))


_CRPT = _ROWS // 32
_CKCH = 56


def _make_count_pass():
  mesh = plsc.VectorSubcoreMesh(core_axis_name="c", subcore_axis_name="s")

  out_type = [
      jax.ShapeDtypeStruct((_NP, _HH), jnp.float32),
      jax.ShapeDtypeStruct((_NP, _HH), jnp.float32),
  ]
  scratch = [
      pltpu.VMEM((_CKCH, _CH), jnp.int32),
      pltpu.VMEM((_CH, _HH), jnp.float32),
      pltpu.VMEM_SHARED((_NP, _HH), jnp.float32),
  ]

  def body(dstr, ones_h, z16, out_a, out_b, didx, ones_v, cacc):
    c = lax.axis_index("c")
    s = lax.axis_index("s")

    pltpu.sync_copy(z16.at[pl.ds(s * _NPT, _NPT)],
                    cacc.at[pl.ds(s * _NPT, _NPT)])
    pltpu.sync_copy(ones_h, ones_v)
    plsc.subcore_barrier()

    row0 = c * (_ROWS // 2) + s * _CRPT

    def batch_body(b, carry):
      r0 = row0 + b * _CKCH
      pltpu.sync_copy(dstr.at[pl.ds(r0, _CKCH)], didx)

      def chunk_body(j, carry2):
        pltpu.sync_copy(ones_v, cacc.at[didx.at[j]], add=True)
        return carry2

      return lax.fori_loop(0, _CKCH, chunk_body, carry)

    lax.fori_loop(0, _CRPT // _CKCH, batch_body, 0)
    plsc.subcore_barrier()

    @pl.when(c == 0)
    def _():
      pltpu.sync_copy(cacc.at[pl.ds(s * _NPT, _NPT)],
                      out_a.at[pl.ds(s * _NPT, _NPT)])

    @pl.when(c == 1)
    def _():
      pltpu.sync_copy(cacc.at[pl.ds(s * _NPT, _NPT)],
                      out_b.at[pl.ds(s * _NPT, _NPT)])

  return pl.kernel(body, mesh=mesh, out_type=out_type,
                   scratch_types=scratch,
                   compiler_params=pltpu.CompilerParams(
                       use_tc_tiling_on_sc=False))


def _combine1_body(al_ref, ar_ref, ca_ref, cb_ref, r1_ref,
                   wl_ref, wr_ref, b2_ref,
                   ql_ref, qr_ref, r2_ref):
  agg = jnp.concatenate([al_ref[...], ar_ref[...]], axis=1)
  cnt = ca_ref[...][:, 0:1] + cb_ref[...][:, 0:1]
  mean = agg / jnp.maximum(cnt, 1.0)
  h1 = jnp.maximum(mean + r1_ref[...], 0.0)
  q2 = jnp.dot(h1, wl_ref[...], preferred_element_type=jnp.float32,
               precision=lax.Precision.HIGHEST)
  ql_ref[...] = q2[:, 0:_HH]
  qr_ref[...] = q2[:, _HH:_HID]
  r2_ref[...] = (jnp.dot(h1, wr_ref[...], preferred_element_type=jnp.float32,
                         precision=lax.Precision.HIGHEST)
                 + b2_ref[...])


def _combine1(al, ar, ca, cb, r1, wl2, wr2, b2r):
  blk_h = pl.BlockSpec((_NB, _HH), lambda i: (i, 0))
  return pl.pallas_call(
      _combine1_body,
      grid=(_GB,),
      in_specs=[
          blk_h, blk_h, blk_h, blk_h,
          pl.BlockSpec((_NB, _HID), lambda i: (i, 0)),
          pl.BlockSpec((_HID, _HID), lambda i: (0, 0)),
          pl.BlockSpec((_HID, _HID), lambda i: (0, 0)),
          pl.BlockSpec((1, _HID), lambda i: (0, 0)),
      ],
      out_specs=(
          blk_h, blk_h,
          pl.BlockSpec((_NB, _HID), lambda i: (i, 0)),
      ),
      out_shape=(
          jax.ShapeDtypeStruct((_NP, _HH), jnp.float32),
          jax.ShapeDtypeStruct((_NP, _HH), jnp.float32),
          jax.ShapeDtypeStruct((_N, _HID), jnp.float32),
      ),
  )(al, ar, ca, cb, r1, wl2, wr2, b2r)


def _combine2_body(al_ref, ar_ref, ca_ref, cb_ref, r2_ref, b_ref,
                   wc_ref, bc_ref,
                   out_ref, gsum_ref, gcnt_ref):
  i = pl.program_id(0)

  @pl.when(i == 0)
  def _():
    gsum_ref[...] = jnp.zeros_like(gsum_ref)
    gcnt_ref[...] = jnp.zeros_like(gcnt_ref)

  agg = jnp.concatenate([al_ref[...], ar_ref[...]], axis=1)
  cnt = ca_ref[...][:, 0:1] + cb_ref[...][:, 0:1]
  mean = agg / jnp.maximum(cnt, 1.0)
  h2 = jnp.maximum(mean + r2_ref[...], 0.0)
  b = b_ref[0, 0, :]
  oh = (b[:, None] == lax.broadcasted_iota(jnp.int32, (_NB, _NG), 1)
        ).astype(jnp.float32)
  gsum_ref[...] += lax.dot_general(
      oh, h2, (((0,), (0,)), ((), ())),
      preferred_element_type=jnp.float32,
      precision=lax.Precision.HIGHEST)
  gcnt_ref[...] += jnp.sum(oh, axis=0)[:, None]

  @pl.when(i == _GB - 1)
  def _():
    hg = gsum_ref[...] / jnp.maximum(gcnt_ref[...], 1.0)
    out_ref[...] = (jnp.dot(hg, wc_ref[...],
                            preferred_element_type=jnp.float32,
                            precision=lax.Precision.HIGHEST)
                    + bc_ref[...])


def _combine2(al, ar, ca, cb, r2, batchr, wc, bcr):
  blk_h = pl.BlockSpec((_NB, _HH), lambda i: (i, 0))
  out, _, _ = pl.pallas_call(
      _combine2_body,
      grid=(_GB,),
      in_specs=[
          blk_h, blk_h, blk_h, blk_h,
          pl.BlockSpec((_NB, _HID), lambda i: (i, 0)),
          pl.BlockSpec((1, 1, _NB), lambda i: (i, 0, 0)),
          pl.BlockSpec((_HID, _NC), lambda i: (0, 0)),
          pl.BlockSpec((1, _NC), lambda i: (0, 0)),
      ],
      out_specs=(
          pl.BlockSpec((_NG, _NC), lambda i: (0, 0)),
          pl.BlockSpec((_NG, _HID), lambda i: (0, 0)),
          pl.BlockSpec((_NG, 1), lambda i: (0, 0)),
      ),
      out_shape=(
          jax.ShapeDtypeStruct((_NG, _NC), jnp.float32),
          jax.ShapeDtypeStruct((_NG, _HID), jnp.float32),
          jax.ShapeDtypeStruct((_NG, 1), jnp.float32),
      ),
  )(al, ar, ca, cb, r2, batchr, wc, bcr)
  return out


def kernel(x, edge_index, batch, shape_emb, color_emb, pos_emb,
           Wl1, Wr1, b1, Wl2, Wr2, b2, Wc, bc):
  f32 = jnp.float32
  x = x.astype(jnp.int32)
  x0r = x[:, 0].reshape(_GB, 1, _NB)
  x1r = x[:, 1].reshape(_GB, 1, _NB)
  x2r = x[:, 2].reshape(_GB, 1, _NB)
  epad = _ROWS * _CH - _E
  srcr = jnp.concatenate(
      [edge_index[0].astype(jnp.int32),
       jnp.zeros((epad,), jnp.int32)]).reshape(_ROWS, _CH)
  dstr = jnp.concatenate(
      [edge_index[1].astype(jnp.int32),
       jnp.full((epad,), _N, jnp.int32)]).reshape(_ROWS, _CH)
  batchr = batch.astype(jnp.int32).reshape(_GB, 1, _NB)

  ws = jnp.concatenate([Wl1[0:16], Wr1[0:16]], axis=1)
  wc2 = jnp.concatenate([Wl1[16:32], Wr1[16:32]], axis=1)
  wp = jnp.concatenate([Wl1[32:48], Wr1[32:48]], axis=1)
  b1r = jnp.concatenate([jnp.zeros((_HID,), f32), b1]).reshape(1, _TW)
  b2r = b2.reshape(1, _HID)
  bcr = bc.reshape(1, _NC)

  ones_h = jnp.ones((_CH, _HH), f32)
  z16 = jnp.zeros((_NP, _HH), f32)

  t = _tables(shape_emb, color_emb, pos_emb, ws, wc2, wp)
  ql, qr, r1 = _embed(x0r, x1r, x2r, t, b1r)
  edge_pass = _make_edge_pass()
  ca, cb = _make_count_pass()(dstr, ones_h, z16)
  al, ar = edge_pass(ql, qr, srcr, dstr, z16)
  q2l, q2r, r2 = _combine1(al, ar, ca, cb, r1, Wl2, Wr2, b2r)
  a2l, a2r = edge_pass(q2l, q2r, srcr, dstr, z16)
  return _combine2(a2l, a2r, ca, cb, r2, batchr, Wc, bcr)


# speedup vs baseline: 1.0072x; 1.0072x over previous
import functools

import jax
import jax.numpy as jnp
from jax import lax
from jax.experimental import pallas as pl
from jax.experimental.pallas import tpu as pltpu
from jax.experimental.pallas import tpu_sc as plsc

_N = 100000
_E = 1600000
_HID = 32
_HH = 16
_NG = 256
_NC = 10
_DIN = 48
_TW = 64

_NB = 2000
_GB = _N // _NB

_NP = 100096
_CH = 128
_ROWS = 12544
_KCH = 16
_RPT = _ROWS // 16
_NPT = _NP // 16
_G = 4
_NGRP = _KCH // _G


def _tables_body(se_ref, ce_ref, pe_ref, ws_ref, wc_ref, wp_ref, t_ref):
  hp = lax.Precision.HIGHEST
  t_ref[0:32, :] = jnp.dot(se_ref[...], ws_ref[...],
                           preferred_element_type=jnp.float32, precision=hp)
  t_ref[32:64, :] = jnp.dot(ce_ref[...], wc_ref[...],
                            preferred_element_type=jnp.float32, precision=hp)
  t_ref[64:192, :] = jnp.dot(pe_ref[...], wp_ref[...],
                             preferred_element_type=jnp.float32, precision=hp)


def _tables(se, ce, pe, ws, wc, wp):
  return pl.pallas_call(
      _tables_body,
      out_shape=jax.ShapeDtypeStruct((192, _TW), jnp.float32),
  )(se, ce, pe, ws, wc, wp)


def _embed_body(x0_ref, x1_ref, x2_ref, t_ref, b1_ref,
                ql_ref, qr_ref, r1_ref):
  x0 = x0_ref[0, 0, :]
  x1 = x1_ref[0, 0, :]
  x2 = x2_ref[0, 0, :]
  i32 = lambda n: lax.broadcasted_iota(jnp.int32, (_NB, n), 1)
  oh = jnp.concatenate([
      (x0[:, None] == i32(32)).astype(jnp.float32),
      (x1[:, None] == i32(32)).astype(jnp.float32),
      (x2[:, None] == i32(128)).astype(jnp.float32),
  ], axis=1)
  qr = jnp.dot(oh, t_ref[...], preferred_element_type=jnp.float32,
               precision=lax.Precision.HIGHEST)
  qr = qr + b1_ref[...]
  ql_ref[...] = qr[:, 0:_HH]
  qr_ref[...] = qr[:, _HH:_HID]
  r1_ref[...] = qr[:, _HID:_TW]


def _embed(x0r, x1r, x2r, t, b1r):
  blk_idx = pl.BlockSpec((1, 1, _NB), lambda i: (i, 0, 0))
  blk_h = pl.BlockSpec((_NB, _HH), lambda i: (i, 0))
  return pl.pallas_call(
      _embed_body,
      grid=(_GB,),
      in_specs=[
          blk_idx, blk_idx, blk_idx,
          pl.BlockSpec((192, _TW), lambda i: (0, 0)),
          pl.BlockSpec((1, _TW), lambda i: (0, 0)),
      ],
      out_specs=(
          blk_h, blk_h,
          pl.BlockSpec((_NB, _HID), lambda i: (i, 0)),
      ),
      out_shape=(
          jax.ShapeDtypeStruct((_NP, _HH), jnp.float32),
          jax.ShapeDtypeStruct((_NP, _HH), jnp.float32),
          jax.ShapeDtypeStruct((_N, _HID), jnp.float32),
      ),
  )(x0r, x1r, x2r, t, b1r)


def _make_edge_pass():
  mesh = plsc.VectorSubcoreMesh(core_axis_name="c", subcore_axis_name="s")

  out_type = [
      jax.ShapeDtypeStruct((_NP, _HH), jnp.float32),
      jax.ShapeDtypeStruct((_NP, _HH), jnp.float32),
  ]
  scratch = [
      pltpu.VMEM((_KCH, _CH), jnp.int32),
      pltpu.VMEM((_KCH, _CH), jnp.int32),
      pltpu.VMEM((3 * _G, _CH, _HH), jnp.float32),
      pltpu.VMEM_SHARED((_NP, _HH), jnp.float32),
      pltpu.SemaphoreType.DMA,
      pltpu.SemaphoreType.DMA,
      pltpu.SemaphoreType.DMA,
      pltpu.SemaphoreType.DMA,
  ]

  def body(ql, qr, srcr, dstr, z16, out_l, out_r,
           sidx, didx, rows, acc, sem_g0, sem_g1, sem_g2, sem_s):
    c = lax.axis_index("c")
    s = lax.axis_index("s")
    sem_g = [sem_g0, sem_g1, sem_g2]

    pltpu.sync_copy(z16.at[pl.ds(s * _NPT, _NPT)],
                    acc.at[pl.ds(s * _NPT, _NPT)])
    plsc.subcore_barrier()

    def run(qtab):
      row0 = s * _RPT

      def fire_gathers(grp, buf):
        for k in range(_G):
          pltpu.async_copy(qtab.at[sidx.at[grp * _G + k]],
                           rows.at[buf * _G + k], sem_g[buf])

      def drain_gathers(buf):
        for k in range(_G):
          pltpu.make_async_copy(qtab.at[sidx.at[k]],
                                rows.at[buf * _G + k], sem_g[buf]).wait()

      def fire_scatters(grp, buf):
        for k in range(_G):
          pltpu.async_copy(rows.at[buf * _G + k],
                           acc.at[didx.at[grp * _G + k]], sem_s, add=True)

      def drain_scatters(n):
        for k in range(n * _G):
          pltpu.make_async_copy(rows.at[k % (3 * _G)],
                                acc.at[didx.at[0]], sem_s).wait()

      def batch_body(b, carry):
        r0 = row0 + b * _KCH
        pltpu.sync_copy(srcr.at[pl.ds(r0, _KCH)], sidx)
        pltpu.sync_copy(dstr.at[pl.ds(r0, _KCH)], didx)
        fire_gathers(0, 0)
        fire_gathers(1, 1)
        fire_gathers(2, 2)
        drain_gathers(0)
        fire_scatters(0, 0)
        drain_gathers(1)
        fire_scatters(1, 1)
        drain_scatters(1)
        fire_gathers(3, 0)
        drain_gathers(2)
        fire_scatters(2, 2)
        drain_gathers(0)
        fire_scatters(3, 0)
        drain_scatters(3)
        return carry

      lax.fori_loop(0, _RPT // _KCH, batch_body, 0)

    @pl.when(c == 0)
    def _():
      run(ql)

    @pl.when(c == 1)
    def _():
      run(qr)

    plsc.subcore_barrier()

    @pl.when(c == 0)
    def _():
      pltpu.sync_copy(acc.at[pl.ds(s * _NPT, _NPT)],
                      out_l.at[pl.ds(s * _NPT, _NPT)])

    @pl.when(c == 1)
    def _():
      pltpu.sync_copy(acc.at[pl.ds(s * _NPT, _NPT)],
                      out_r.at[pl.ds(s * _NPT, _NPT)])

  return pl.kernel(body, mesh=mesh, out_type=out_type,
                   scratch_types=scratch,
                   compiler_params=pltpu.CompilerParams(
                       use_tc_tiling_on_sc=False))


_CRPT = _ROWS // 32
_CKCH = 56


def _make_count_pass():
  mesh = plsc.VectorSubcoreMesh(core_axis_name="c", subcore_axis_name="s")

  out_type = [
      jax.ShapeDtypeStruct((_NP, _HH), jnp.float32),
      jax.ShapeDtypeStruct((_NP, _HH), jnp.float32),
  ]
  scratch = [
      pltpu.VMEM((_CKCH, _CH), jnp.int32),
      pltpu.VMEM((_CH, _HH), jnp.float32),
      pltpu.VMEM_SHARED((_NP, _HH), jnp.float32),
  ]

  def body(dstr, ones_h, z16, out_a, out_b, didx, ones_v, cacc):
    c = lax.axis_index("c")
    s = lax.axis_index("s")

    pltpu.sync_copy(z16.at[pl.ds(s * _NPT, _NPT)],
                    cacc.at[pl.ds(s * _NPT, _NPT)])
    pltpu.sync_copy(ones_h, ones_v)
    plsc.subcore_barrier()

    row0 = c * (_ROWS // 2) + s * _CRPT

    def batch_body(b, carry):
      r0 = row0 + b * _CKCH
      pltpu.sync_copy(dstr.at[pl.ds(r0, _CKCH)], didx)

      def chunk_body(j, carry2):
        pltpu.sync_copy(ones_v, cacc.at[didx.at[j]], add=True)
        return carry2

      return lax.fori_loop(0, _CKCH, chunk_body, carry)

    lax.fori_loop(0, _CRPT // _CKCH, batch_body, 0)
    plsc.subcore_barrier()

    @pl.when(c == 0)
    def _():
      pltpu.sync_copy(cacc.at[pl.ds(s * _NPT, _NPT)],
                      out_a.at[pl.ds(s * _NPT, _NPT)])

    @pl.when(c == 1)
    def _():
      pltpu.sync_copy(cacc.at[pl.ds(s * _NPT, _NPT)],
                      out_b.at[pl.ds(s * _NPT, _NPT)])

  return pl.kernel(body, mesh=mesh, out_type=out_type,
                   scratch_types=scratch,
                   compiler_params=pltpu.CompilerParams(
                       use_tc_tiling_on_sc=False))


def _combine1_body(al_ref, ar_ref, ca_ref, cb_ref, r1_ref,
                   wl_ref, wr_ref, b2_ref,
                   ql_ref, qr_ref, r2_ref):
  agg = jnp.concatenate([al_ref[...], ar_ref[...]], axis=1)
  cnt = ca_ref[...][:, 0:1] + cb_ref[...][:, 0:1]
  mean = agg / jnp.maximum(cnt, 1.0)
  h1 = jnp.maximum(mean + r1_ref[...], 0.0)
  q2 = jnp.dot(h1, wl_ref[...], preferred_element_type=jnp.float32,
               precision=lax.Precision.HIGHEST)
  ql_ref[...] = q2[:, 0:_HH]
  qr_ref[...] = q2[:, _HH:_HID]
  r2_ref[...] = (jnp.dot(h1, wr_ref[...], preferred_element_type=jnp.float32,
                         precision=lax.Precision.HIGHEST)
                 + b2_ref[...])


def _combine1(al, ar, ca, cb, r1, wl2, wr2, b2r):
  blk_h = pl.BlockSpec((_NB, _HH), lambda i: (i, 0))
  return pl.pallas_call(
      _combine1_body,
      grid=(_GB,),
      in_specs=[
          blk_h, blk_h, blk_h, blk_h,
          pl.BlockSpec((_NB, _HID), lambda i: (i, 0)),
          pl.BlockSpec((_HID, _HID), lambda i: (0, 0)),
          pl.BlockSpec((_HID, _HID), lambda i: (0, 0)),
          pl.BlockSpec((1, _HID), lambda i: (0, 0)),
      ],
      out_specs=(
          blk_h, blk_h,
          pl.BlockSpec((_NB, _HID), lambda i: (i, 0)),
      ),
      out_shape=(
          jax.ShapeDtypeStruct((_NP, _HH), jnp.float32),
          jax.ShapeDtypeStruct((_NP, _HH), jnp.float32),
          jax.ShapeDtypeStruct((_N, _HID), jnp.float32),
      ),
  )(al, ar, ca, cb, r1, wl2, wr2, b2r)


def _combine2_body(al_ref, ar_ref, ca_ref, cb_ref, r2_ref, b_ref,
                   wc_ref, bc_ref,
                   out_ref, gsum_ref, gcnt_ref):
  i = pl.program_id(0)

  @pl.when(i == 0)
  def _():
    gsum_ref[...] = jnp.zeros_like(gsum_ref)
    gcnt_ref[...] = jnp.zeros_like(gcnt_ref)

  agg = jnp.concatenate([al_ref[...], ar_ref[...]], axis=1)
  cnt = ca_ref[...][:, 0:1] + cb_ref[...][:, 0:1]
  mean = agg / jnp.maximum(cnt, 1.0)
  h2 = jnp.maximum(mean + r2_ref[...], 0.0)
  b = b_ref[0, 0, :]
  oh = (b[:, None] == lax.broadcasted_iota(jnp.int32, (_NB, _NG), 1)
        ).astype(jnp.float32)
  gsum_ref[...] += lax.dot_general(
      oh, h2, (((0,), (0,)), ((), ())),
      preferred_element_type=jnp.float32,
      precision=lax.Precision.HIGHEST)
  gcnt_ref[...] += jnp.sum(oh, axis=0)[:, None]

  @pl.when(i == _GB - 1)
  def _():
    hg = gsum_ref[...] / jnp.maximum(gcnt_ref[...], 1.0)
    out_ref[...] = (jnp.dot(hg, wc_ref[...],
                            preferred_element_type=jnp.float32,
                            precision=lax.Precision.HIGHEST)
                    + bc_ref[...])


def _combine2(al, ar, ca, cb, r2, batchr, wc, bcr):
  blk_h = pl.BlockSpec((_NB, _HH), lambda i: (i, 0))
  out, _, _ = pl.pallas_call(
      _combine2_body,
      grid=(_GB,),
      in_specs=[
          blk_h, blk_h, blk_h, blk_h,
          pl.BlockSpec((_NB, _HID), lambda i: (i, 0)),
          pl.BlockSpec((1, 1, _NB), lambda i: (i, 0, 0)),
          pl.BlockSpec((_HID, _NC), lambda i: (0, 0)),
          pl.BlockSpec((1, _NC), lambda i: (0, 0)),
      ],
      out_specs=(
          pl.BlockSpec((_NG, _NC), lambda i: (0, 0)),
          pl.BlockSpec((_NG, _HID), lambda i: (0, 0)),
          pl.BlockSpec((_NG, 1), lambda i: (0, 0)),
      ),
      out_shape=(
          jax.ShapeDtypeStruct((_NG, _NC), jnp.float32),
          jax.ShapeDtypeStruct((_NG, _HID), jnp.float32),
          jax.ShapeDtypeStruct((_NG, 1), jnp.float32),
      ),
  )(al, ar, ca, cb, r2, batchr, wc, bcr)
  return out


def kernel(x, edge_index, batch, shape_emb, color_emb, pos_emb,
           Wl1, Wr1, b1, Wl2, Wr2, b2, Wc, bc):
  f32 = jnp.float32
  x = x.astype(jnp.int32)
  x0r = x[:, 0].reshape(_GB, 1, _NB)
  x1r = x[:, 1].reshape(_GB, 1, _NB)
  x2r = x[:, 2].reshape(_GB, 1, _NB)
  epad = _ROWS * _CH - _E
  srcr = jnp.concatenate(
      [edge_index[0].astype(jnp.int32),
       jnp.zeros((epad,), jnp.int32)]).reshape(_ROWS, _CH)
  dstr = jnp.concatenate(
      [edge_index[1].astype(jnp.int32),
       jnp.full((epad,), _N, jnp.int32)]).reshape(_ROWS, _CH)
  batchr = batch.astype(jnp.int32).reshape(_GB, 1, _NB)

  ws = jnp.concatenate([Wl1[0:16], Wr1[0:16]], axis=1)
  wc2 = jnp.concatenate([Wl1[16:32], Wr1[16:32]], axis=1)
  wp = jnp.concatenate([Wl1[32:48], Wr1[32:48]], axis=1)
  b1r = jnp.concatenate([jnp.zeros((_HID,), f32), b1]).reshape(1, _TW)
  b2r = b2.reshape(1, _HID)
  bcr = bc.reshape(1, _NC)

  ones_h = jnp.ones((_CH, _HH), f32)
  z16 = jnp.zeros((_NP, _HH), f32)

  t = _tables(shape_emb, color_emb, pos_emb, ws, wc2, wp)
  ql, qr, r1 = _embed(x0r, x1r, x2r, t, b1r)
  edge_pass = _make_edge_pass()
  ca, cb = _make_count_pass()(dstr, ones_h, z16)
  al, ar = edge_pass(ql, qr, srcr, dstr, z16)
  q2l, q2r, r2 = _combine1(al, ar, ca, cb, r1, Wl2, Wr2, b2r)
  a2l, a2r = edge_pass(q2l, q2r, srcr, dstr, z16)
  return _combine2(a2l, a2r, ca, cb, r2, batchr, Wc, bcr)
